# baseline (speedup 1.0000x reference)
import functools

import jax
import jax.numpy as jnp
from jax import lax
from jax.experimental import pallas as pl
from jax.experimental.pallas import tpu as pltpu
from jax.experimental.pallas import tpu_sc as plsc

L = 16
CH = 256


@functools.lru_cache(maxsize=None)
def _build(B, D):
    info = plsc.get_sparse_core_info()
    NC, NS = info.num_cores, info.num_subcores
    NW = NC * NS
    assert B % (8 * NW) == 0 and D % L == 0
    bpw = B // NW
    ng = bpw // L
    W = 2 * D
    n_ch = bpw // CH
    ng_ch = CH // L

    mesh = plsc.VectorSubcoreMesh(core_axis_name="c", subcore_axis_name="s")

    @functools.partial(
        pl.kernel,
        mesh=mesh,
        compiler_params=pltpu.CompilerParams(needs_layout_passes=False),
        out_type=(
            jax.ShapeDtypeStruct((B,), jnp.float32),
            jax.ShapeDtypeStruct((B,), jnp.float32),
        ),
        scratch_types=[
            pltpu.VMEM((bpw,), jnp.int32),
            pltpu.VMEM((bpw,), jnp.int32),
            pltpu.VMEM((bpw,), jnp.int32),
            pltpu.VMEM((bpw,), jnp.int32),
            pltpu.VMEM((CH, W), jnp.float32),
            pltpu.VMEM((CH, W), jnp.float32),
            pltpu.VMEM((bpw,), jnp.float32),
            pltpu.VMEM((bpw,), jnp.float32),
            pltpu.VMEM((bpw,), jnp.float32),
            pltpu.VMEM((bpw,), jnp.float32),
            pltpu.VMEM((2 * L,), jnp.float32),
            pltpu.SemaphoreType.DMA,
            pltpu.SemaphoreType.DMA,
            pltpu.SemaphoreType.DMA,
            pltpu.SemaphoreType.DMA,
        ],
    )
    def k(users, items, uW, iW, user_b, prod_b, scal,
          pred_out, log_out,
          ur_i, ir_i, ub_i, pb_i, uv_v, iv_v, ub_v, pb_v, log_v, pred_v,
          sc_v, s0, s1, s2, s3):
        wid = lax.axis_index("s") * NC + lax.axis_index("c")
        base = wid * bpw

        pltpu.sync_copy(users.at[pl.ds(base, bpw)], ur_i)
        pltpu.sync_copy(items.at[pl.ds(base, bpw)], pb_i)
        pltpu.sync_copy(scal, sc_v)

        def _prep(j, carry):
            sl = pl.ds(j * L, L)
            u = ur_i[sl]
            it = pb_i[sl]
            ub_i[sl] = u + u
            ir_i[sl] = lax.shift_right_logical(it, 1)
            return carry

        lax.fori_loop(0, ng, _prep, 0)

        cp2 = pltpu.async_copy(user_b.at[ub_i], ub_v, s2)
        cp3 = pltpu.async_copy(prod_b.at[pb_i], pb_v, s3)
        cp2.wait()
        cp3.wait()

        alpha_s = sc_v[pl.ds(0, L)]
        g_s = sc_v[pl.ds(L, L)]

        for c in range(n_ch):
            cbase = c * CH
            cp0 = pltpu.async_copy(uW.at[ur_i.at[pl.ds(cbase, CH)]],
                                   uv_v, s0)
            cp1 = pltpu.async_copy(iW.at[ir_i.at[pl.ds(cbase, CH)]],
                                   iv_v, s1)
            cp0.wait()
            cp1.wait()

            def _blk(b, carry):
                off = b * L
                sl = pl.ds(cbase + off, L)
                rows = off + lax.iota(jnp.int32, L)
                ioff = (pb_i[sl] & 1) * D
                acc = jnp.zeros((L,), jnp.float32)
                for dd in range(D):
                    ucols = jnp.full((L,), dd, jnp.int32)
                    u = plsc.load_gather(uv_v, [rows, ucols])
                    iv = plsc.load_gather(iv_v, [rows, ioff + dd])
                    acc = acc + u * iv
                logit = alpha_s * acc + ub_v[sl] + pb_v[sl] + g_s
                log_v[sl] = logit
                pred_v[sl] = 1.0 / (1.0 + jnp.exp(-logit))
                return carry

            lax.fori_loop(0, ng_ch, _blk, 0)

        pltpu.sync_copy(log_v, log_out.at[pl.ds(base, bpw)])
        pltpu.sync_copy(pred_v, pred_out.at[pl.ds(base, bpw)])

    return k


def kernel(users, items, user_emb, item_emb, alpha, global_bias, user_b, prod_b):
    B = users.shape[0]
    V, D = user_emb.shape
    users = users.astype(jnp.int32)
    items = items.astype(jnp.int32)
    uW = user_emb.reshape(V // 2, 2 * D)
    iW = item_emb.reshape(V // 2, 2 * D)
    scal = jnp.concatenate([
        jnp.broadcast_to(alpha.astype(jnp.float32), (L,)),
        jnp.broadcast_to(global_bias.astype(jnp.float32), (L,)),
    ])
    pred, logits = _build(B, D)(users, items, uW, iW, user_b, prod_b, scal)
    return pred.reshape(B, 1), logits.reshape(B, 1)


# speedup vs baseline: 2.8045x; 2.8045x over previous
import functools

import jax
import jax.numpy as jnp
from jax import lax
from jax.experimental import pallas as pl
from jax.experimental.pallas import tpu as pltpu
from jax.experimental.pallas import tpu_sc as plsc

L = 16
TW = 128
NBUF = 4


@functools.lru_cache(maxsize=None)
def _build(B, D, V):
    info = plsc.get_sparse_core_info()
    NC, NS = info.num_cores, info.num_subcores
    NW = NC * NS
    assert B % (L * NW) == 0 and D % L == 0
    bpw = B // NW
    ng = bpw // L

    mesh = plsc.VectorSubcoreMesh(core_axis_name="c", subcore_axis_name="s")

    @functools.partial(
        pl.kernel,
        mesh=mesh,
        compiler_params=pltpu.CompilerParams(needs_layout_passes=False),
        out_type=(
            jax.ShapeDtypeStruct((B,), jnp.float32),
            jax.ShapeDtypeStruct((B,), jnp.float32),
        ),
        scratch_types=[
            pltpu.VMEM((bpw + 2 * L,), jnp.int32),
            pltpu.VMEM((bpw + 2 * L,), jnp.int32),
            pltpu.VMEM((bpw,), jnp.int32),
            pltpu.VMEM((bpw,), jnp.int32),
            [pltpu.VMEM((D, TW), jnp.float32) for _ in range(NBUF)],
            [pltpu.VMEM((D, TW), jnp.float32) for _ in range(NBUF)],
            pltpu.VMEM((L * L,), jnp.float32),
            pltpu.VMEM((bpw,), jnp.float32),
            pltpu.VMEM((bpw,), jnp.float32),
            pltpu.VMEM((bpw,), jnp.float32),
            pltpu.VMEM((bpw,), jnp.float32),
            pltpu.VMEM((2 * L,), jnp.float32),
            [pltpu.SemaphoreType.DMA for _ in range(NBUF)],
            [pltpu.SemaphoreType.DMA for _ in range(NBUF)],
            pltpu.SemaphoreType.DMA,
            pltpu.SemaphoreType.DMA,
        ],
    )
    def k(users, items, uT, iT, user_b, prod_b, scal,
          pred_out, log_out,
          usm, ism, ub_i, pb_i, ubufs, ibufs, dots_v, ub_v, pb_v,
          log_v, pred_v, sc_v, usems, isems, s_ub, s_pb):
        wid = lax.axis_index("s") * NC + lax.axis_index("c")
        base = wid * bpw

        pltpu.sync_copy(users.at[pl.ds(base, bpw)], ub_i)
        pltpu.sync_copy(items.at[pl.ds(base, bpw)], pb_i)
        pltpu.sync_copy(users.at[pl.ds(base, bpw)], usm.at[pl.ds(0, bpw)])
        pltpu.sync_copy(items.at[pl.ds(base, bpw)], ism.at[pl.ds(0, bpw)])
        lastu = usm[pl.ds(bpw - L, L)][L - 1]
        lasti = ism[pl.ds(bpw - L, L)][L - 1]
        usm[pl.ds(bpw, L)] = jnp.full((L,), lastu, jnp.int32)
        usm[pl.ds(bpw + L, L)] = jnp.full((L,), lastu, jnp.int32)
        ism[pl.ds(bpw, L)] = jnp.full((L,), lasti, jnp.int32)
        ism[pl.ds(bpw + L, L)] = jnp.full((L,), lasti, jnp.int32)
        pltpu.sync_copy(scal, sc_v)

        def _prep(j, carry):
            sl = pl.ds(j * L, L)
            u = ub_i[sl]
            ub_i[sl] = u + u
            return carry

        lax.fori_loop(0, ng, _prep, 0)

        cpb0 = pltpu.async_copy(user_b.at[ub_i], ub_v, s_ub)
        cpb1 = pltpu.async_copy(prod_b.at[pb_i], pb_v, s_pb)
        cpb0.wait()
        cpb1.wait()

        def _fire(e, slot):
            u0 = usm[pl.ds(e, L)][0]
            i0 = ism[pl.ds(e, L)][0]
            cu = lax.shift_right_logical(u0, 6)
            ci = lax.shift_right_logical(i0, 7)
            ou = pl.multiple_of(cu * TW, TW)
            oi = pl.multiple_of(ci * TW, TW)
            pltpu.async_copy(uT.at[:, pl.ds(ou, TW)], ubufs[slot],
                             usems[slot])
            pltpu.async_copy(iT.at[:, pl.ds(oi, TW)], ibufs[slot],
                             isems[slot])

        def _wait(slot):
            pltpu.make_async_copy(uT.at[:, pl.ds(0, TW)], ubufs[slot],
                                  usems[slot]).wait()
            pltpu.make_async_copy(iT.at[:, pl.ds(0, TW)], ibufs[slot],
                                  isems[slot]).wait()

        for p in range(NBUF):
            _fire(p, p)

        rows16 = lax.iota(jnp.int32, L)
        alpha_s = sc_v[pl.ds(0, L)]
        g_s = sc_v[pl.ds(L, L)]

        def _group(g, carry):
            for t in range(L):
                e = g * L + t
                slot = t % NBUF
                _wait(slot)
                u0 = usm[pl.ds(e, L)][0]
                i0 = ism[pl.ds(e, L)][0]
                ju = (u0 & 63) * 2
                ji = i0 & 127
                ucols = jnp.full((L,), ju, jnp.int32)
                icols = jnp.full((L,), ji, jnp.int32)
                s_e = jnp.zeros((L,), jnp.float32)
                for kk in range(D // L):
                    r = kk * L + rows16
                    uu = plsc.load_gather(ubufs[slot], [r, ucols])
                    ii = plsc.load_gather(ibufs[slot], [r, icols])
                    s_e = s_e + uu * ii
                dots_v[pl.ds(t * L, L)] = s_e
                _fire(e + NBUF, slot)
            dot = jnp.zeros((L,), jnp.float32)
            for c in range(L):
                dot = dot + plsc.load_gather(dots_v, [rows16 * L + c])
            sl = pl.ds(g * L, L)
            logit = alpha_s * dot + ub_v[sl] + pb_v[sl] + g_s
            log_v[sl] = logit
            pred_v[sl] = 1.0 / (1.0 + jnp.exp(-logit))
            return carry

        lax.fori_loop(0, ng, _group, 0)

        for p in range(NBUF):
            _wait(p)

        pltpu.sync_copy(log_v, log_out.at[pl.ds(base, bpw)])
        pltpu.sync_copy(pred_v, pred_out.at[pl.ds(base, bpw)])

    return k


def kernel(users, items, user_emb, item_emb, alpha, global_bias, user_b, prod_b):
    B = users.shape[0]
    V, D = user_emb.shape
    users = users.astype(jnp.int32)
    items = items.astype(jnp.int32)
    uT = user_emb.T
    iT = item_emb.T
    scal = jnp.concatenate([
        jnp.broadcast_to(alpha.astype(jnp.float32), (L,)),
        jnp.broadcast_to(global_bias.astype(jnp.float32), (L,)),
    ])
    pred, logits = _build(B, D, V)(users, items, uT, iT, user_b, prod_b, scal)
    return pred.reshape(B, 1), logits.reshape(B, 1)
